# baseline (speedup 1.0000x reference)
import functools

import jax
import jax.numpy as jnp
from jax import lax
from jax.experimental import pallas as pl
from jax.experimental.pallas import tpu as pltpu
from jax.experimental.pallas import tpu_sc as plsc

NC = 2
NS = 16
L = 16
NU = NC * NS // 2

VOCAB = 1000000
B = 16384
EMB = 16
RPU = B // NU
GB = 16
NGRP = RPU // GB
NSLOT = 3

_mesh = plsc.VectorSubcoreMesh(core_axis_name="c", subcore_axis_name="s")


@functools.partial(
    pl.kernel,
    out_type=jax.ShapeDtypeStruct((B,), jnp.float32),
    mesh=_mesh,
    compiler_params=pltpu.CompilerParams(needs_layout_passes=False, use_tc_tiling_on_sc=True),
    scratch_types=[
        pltpu.VMEM((RPU,), jnp.int32),
        pltpu.VMEM((RPU,), jnp.int32),
        pltpu.VMEM((NSLOT * GB, 8, 128), jnp.float32),
        pltpu.VMEM((NSLOT * GB, 8, 128), jnp.float32),
        pltpu.VMEM((RPU,), jnp.float32),
        pltpu.VMEM((RPU,), jnp.float32),
        pltpu.VMEM((RPU // 2,), jnp.float32),
        pltpu.VMEM_SHARED((NS * RPU,), jnp.float32),
        pltpu.SemaphoreType.DMA,
    ],
)
def _w2v_sc(tid_hbm, cid_hbm, tT_hbm, cT_hbm, out_hbm,
            vt, vc, twin, cwin, part, pbuf, outbuf, shared, sem):
    cidx = lax.axis_index("c")
    sidx = lax.axis_index("s")
    half = jnp.bitwise_and(sidx, 1)
    unit = cidx * (NS // 2) + jnp.right_shift(sidx, 1)
    base = unit * RPU
    dh = pl.multiple_of(half * 8, 8)

    pltpu.sync_copy(tid_hbm.at[pl.ds(base, RPU)], vt)
    pltpu.sync_copy(cid_hbm.at[pl.ds(base, RPU)], vc)

    lane = lax.iota(jnp.int32, L)

    def fire(g, slot):
        gb = pl.multiple_of(g * GB, GB)
        tids = vt[pl.ds(gb, GB)]
        cids = vc[pl.ds(gb, GB)]
        wb = slot * GB
        for i in range(GB):
            t0 = pl.multiple_of((tids[i] >> 7) << 7, 128)
            c0 = pl.multiple_of((cids[i] >> 7) << 7, 128)
            pltpu.async_copy(tT_hbm.at[pl.ds(dh, 8), pl.ds(t0, 128)],
                             twin.at[wb + i], sem)
            pltpu.async_copy(cT_hbm.at[pl.ds(dh, 8), pl.ds(c0, 128)],
                             cwin.at[wb + i], sem)

    fire(0, 0)
    fire(1, 1)

    def group_body(g, carry):
        pltpu.make_async_copy(tT_hbm.at[pl.ds(0, 8), pl.ds(0, 128)],
                              twin.at[pl.ds(0, GB)], sem).wait()
        pltpu.make_async_copy(tT_hbm.at[pl.ds(0, 8), pl.ds(0, 128)],
                              cwin.at[pl.ds(0, GB)], sem).wait()

        @pl.when(g + 2 < NGRP)
        def _():
            fire(g + 2, lax.rem(g + 2, NSLOT))

        slot = lax.rem(g, NSLOT)
        gb = pl.multiple_of(g * GB, GB)
        tids = vt[pl.ds(gb, GB)]
        cids = vc[pl.ds(gb, GB)]
        tcol = jnp.bitwise_and(tids, 127)
        ccol = jnp.bitwise_and(cids, 127)
        wvec = slot * GB + lane
        acc = jnp.zeros((L,), jnp.float32)
        for d in range(EMB // 2):
            dv = jnp.full((L,), d, jnp.int32)
            tv = plsc.load_gather(twin, [wvec, dv, tcol])
            cv = plsc.load_gather(cwin, [wvec, dv, ccol])
            acc = acc + tv * cv
        part[pl.ds(gb, GB)] = acc
        return carry

    lax.fori_loop(0, NGRP, group_body, 0)

    pltpu.sync_copy(part, shared.at[pl.ds(sidx * RPU, RPU)])
    plsc.subcore_barrier()
    pltpu.sync_copy(shared.at[pl.ds(jnp.bitwise_xor(sidx, 1) * RPU, RPU)], pbuf)

    myoff = pl.multiple_of(half * (RPU // 2), RPU // 2)
    for v in range(RPU // 2 // L):
        sl = pl.ds(pl.multiple_of(myoff + v * L, L), L)
        x = part[sl] + pbuf[sl]
        e = jnp.exp(-jnp.abs(x))
        s = e / (2.0 + e)
        s2 = s * s
        poly = 1.0 + s2 * (1.0 / 3.0 + s2 * (1.0 / 5.0 + s2 * (1.0 / 7.0 + s2 * (1.0 / 9.0))))
        outbuf[pl.ds(v * L, L)] = jnp.minimum(x, 0.0) - 2.0 * s * poly

    pltpu.sync_copy(outbuf, out_hbm.at[pl.ds(base + myoff, RPU // 2)])


def kernel(target_ids, context_ids, target_embeddings, context_embeddings):
    return _w2v_sc(target_ids.astype(jnp.int32), context_ids.astype(jnp.int32),
                   target_embeddings.T, context_embeddings.T)


# speedup vs baseline: 1.0060x; 1.0060x over previous
import functools

import jax
import jax.numpy as jnp
from jax import lax
from jax.experimental import pallas as pl
from jax.experimental.pallas import tpu as pltpu
from jax.experimental.pallas import tpu_sc as plsc

NC = 2
NS = 16
L = 16
NU = NC * NS // 2

VOCAB = 1000000
B = 16384
EMB = 16
RPU = B // NU
GB = 16
NGRP = RPU // GB
NSLOT = 3

_mesh = plsc.VectorSubcoreMesh(core_axis_name="c", subcore_axis_name="s")


@functools.partial(
    pl.kernel,
    out_type=jax.ShapeDtypeStruct((B,), jnp.float32),
    mesh=_mesh,
    compiler_params=pltpu.CompilerParams(needs_layout_passes=False, use_tc_tiling_on_sc=True),
    scratch_types=[
        pltpu.VMEM((RPU,), jnp.int32),
        pltpu.VMEM((RPU,), jnp.int32),
        pltpu.VMEM((NSLOT * GB, 8, 128), jnp.float32),
        pltpu.VMEM((NSLOT * GB, 8, 128), jnp.float32),
        pltpu.VMEM((RPU,), jnp.float32),
        pltpu.VMEM((RPU,), jnp.float32),
        pltpu.VMEM((RPU // 2,), jnp.float32),
        pltpu.VMEM_SHARED((NS * RPU,), jnp.float32),
        pltpu.SemaphoreType.DMA,
    ],
)
def _w2v_sc(tid_hbm, cid_hbm, tT_hbm, cT_hbm, out_hbm,
            vt, vc, twin, cwin, part, pbuf, outbuf, shared, sem):
    cidx = lax.axis_index("c")
    sidx = lax.axis_index("s")
    half = jnp.bitwise_and(sidx, 1)
    unit = cidx * (NS // 2) + jnp.right_shift(sidx, 1)
    base = unit * RPU
    dh = pl.multiple_of(half * 8, 8)

    pltpu.sync_copy(tid_hbm.at[pl.ds(base, RPU)], vt)
    pltpu.sync_copy(cid_hbm.at[pl.ds(base, RPU)], vc)

    lane = lax.iota(jnp.int32, L)

    def fire(g, slot):
        gb = pl.multiple_of(g * GB, GB)
        tids = vt[pl.ds(gb, GB)]
        cids = vc[pl.ds(gb, GB)]
        wb = slot * GB
        for i in range(GB):
            t0 = pl.multiple_of((tids[i] >> 7) << 7, 128)
            c0 = pl.multiple_of((cids[i] >> 7) << 7, 128)
            pltpu.async_copy(tT_hbm.at[pl.ds(dh, 8), pl.ds(t0, 128)],
                             twin.at[wb + i], sem)
            pltpu.async_copy(cT_hbm.at[pl.ds(dh, 8), pl.ds(c0, 128)],
                             cwin.at[wb + i], sem)

    fire(0, 0)
    fire(1, 1)

    def group_body(g, carry):
        pltpu.make_async_copy(tT_hbm.at[:, pl.ds(0, GB * 64)],
                              twin.at[pl.ds(0, GB)], sem).wait()
        pltpu.make_async_copy(cT_hbm.at[:, pl.ds(0, GB * 64)],
                              cwin.at[pl.ds(0, GB)], sem).wait()

        @pl.when(g + 2 < NGRP)
        def _():
            fire(g + 2, lax.rem(g + 2, NSLOT))

        slot = lax.rem(g, NSLOT)
        gb = pl.multiple_of(g * GB, GB)
        tids = vt[pl.ds(gb, GB)]
        cids = vc[pl.ds(gb, GB)]
        tcol = jnp.bitwise_and(tids, 127)
        ccol = jnp.bitwise_and(cids, 127)
        wvec = slot * GB + lane
        acc = jnp.zeros((L,), jnp.float32)
        for d in range(EMB // 2):
            dv = jnp.full((L,), d, jnp.int32)
            tv = plsc.load_gather(twin, [wvec, dv, tcol])
            cv = plsc.load_gather(cwin, [wvec, dv, ccol])
            acc = acc + tv * cv
        part[pl.ds(gb, GB)] = acc
        return carry

    lax.fori_loop(0, NGRP, group_body, 0)

    pltpu.sync_copy(part, shared.at[pl.ds(sidx * RPU, RPU)])
    plsc.subcore_barrier()
    pltpu.sync_copy(shared.at[pl.ds(jnp.bitwise_xor(sidx, 1) * RPU, RPU)], pbuf)

    myoff = pl.multiple_of(half * (RPU // 2), RPU // 2)
    for v in range(RPU // 2 // L):
        sl = pl.ds(pl.multiple_of(myoff + v * L, L), L)
        x = part[sl] + pbuf[sl]
        e = jnp.exp(-jnp.abs(x))
        s = e / (2.0 + e)
        s2 = s * s
        poly = 1.0 + s2 * (1.0 / 3.0 + s2 * (1.0 / 5.0 + s2 * (1.0 / 7.0 + s2 * (1.0 / 9.0))))
        outbuf[pl.ds(v * L, L)] = jnp.minimum(x, 0.0) - 2.0 * s * poly

    pltpu.sync_copy(outbuf, out_hbm.at[pl.ds(base + myoff, RPU // 2)])


def kernel(target_ids, context_ids, target_embeddings, context_embeddings):
    return _w2v_sc(target_ids.astype(jnp.int32), context_ids.astype(jnp.int32),
                   target_embeddings.T, context_embeddings.T)
